# Initial kernel scaffold; baseline (speedup 1.0000x reference)
import jax
import jax.numpy as jnp
from jax.experimental import pallas as pl


def kernel(agent_hist, lane_nodes, edge_index_aa, edge_index_al, params):
    raise NotImplementedError("write your pallas kernel here")


# speedup vs baseline: 1.3822x; 1.3822x over previous
import functools

import jax
import jax.numpy as jnp
from jax import lax
from jax.experimental import pallas as pl

H = 64
N_NODES = 50000
BLK = 2000


def _lstm_body(x_ref, wih_ref, whh_ref, b_ref, out_ref, *, T, mean):
    B = x_ref.shape[1]
    h0 = jnp.zeros((B, H), jnp.float32)

    def step(t, carry):
        h, c, acc = carry
        xt = x_ref[t]
        gates = jnp.dot(xt, wih_ref[...], preferred_element_type=jnp.float32)
        gates += jnp.dot(h, whh_ref[...], preferred_element_type=jnp.float32)
        gates += b_ref[...]
        i = jax.nn.sigmoid(gates[:, 0:H])
        f = jax.nn.sigmoid(gates[:, H:2 * H])
        g = jnp.tanh(gates[:, 2 * H:3 * H])
        o = jax.nn.sigmoid(gates[:, 3 * H:4 * H])
        c = f * c + i * g
        h = o * jnp.tanh(c)
        return h, c, acc + h

    h, c, acc = lax.fori_loop(0, T, step, (h0, h0, h0))
    out_ref[...] = (acc * (1.0 / T)) if mean else h


def _lstm(x, Wih, Whh, bias, mean):
    T, N, Dp = x.shape
    grid = (N // BLK,)
    return pl.pallas_call(
        functools.partial(_lstm_body, T=T, mean=mean),
        grid=grid,
        in_specs=[
            pl.BlockSpec((T, BLK, Dp), lambda i: (0, i, 0)),
            pl.BlockSpec((Dp, 4 * H), lambda i: (0, 0)),
            pl.BlockSpec((H, 4 * H), lambda i: (0, 0)),
            pl.BlockSpec((1, 4 * H), lambda i: (0, 0)),
        ],
        out_specs=pl.BlockSpec((BLK, H), lambda i: (i, 0)),
        out_shape=jax.ShapeDtypeStruct((N, H), jnp.float32),
    )(x, Wih, Whh, bias)


def _pre_body(x_ref, w_ref, a_ref, hs_ref, att_ref):
    hs = jnp.dot(x_ref[...], w_ref[...], preferred_element_type=jnp.float32)
    hs_ref[...] = hs
    att_ref[...] = jnp.dot(hs, a_ref[...], preferred_element_type=jnp.float32)


def _gat_pre(x, W, Acomb):
    N = x.shape[0]
    return pl.pallas_call(
        _pre_body,
        grid=(N // BLK,),
        in_specs=[
            pl.BlockSpec((BLK, H), lambda i: (i, 0)),
            pl.BlockSpec((H, H), lambda i: (0, 0)),
            pl.BlockSpec((H, 8), lambda i: (0, 0)),
        ],
        out_specs=[
            pl.BlockSpec((BLK, H), lambda i: (i, 0)),
            pl.BlockSpec((BLK, 8), lambda i: (i, 0)),
        ],
        out_shape=[
            jax.ShapeDtypeStruct((N, H), jnp.float32),
            jax.ShapeDtypeStruct((N, 8), jnp.float32),
        ],
    )(x, W, Acomb)


def _pre2_body(xs_ref, xd_ref, ws_ref, wd_ref, as_ref, ad_ref, hs_ref, att_ref):
    hs = jnp.dot(xs_ref[...], ws_ref[...], preferred_element_type=jnp.float32)
    hd = jnp.dot(xd_ref[...], wd_ref[...], preferred_element_type=jnp.float32)
    hs_ref[...] = hs
    att_ref[...] = (
        jnp.dot(hs, as_ref[...], preferred_element_type=jnp.float32)
        + jnp.dot(hd, ad_ref[...], preferred_element_type=jnp.float32))


def _gat_pre2(x_src, x_dst, Wsrc, Wdst, As, Ad):
    N = x_src.shape[0]
    return pl.pallas_call(
        _pre2_body,
        grid=(N // BLK,),
        in_specs=[
            pl.BlockSpec((BLK, H), lambda i: (i, 0)),
            pl.BlockSpec((BLK, H), lambda i: (i, 0)),
            pl.BlockSpec((H, H), lambda i: (0, 0)),
            pl.BlockSpec((H, H), lambda i: (0, 0)),
            pl.BlockSpec((H, 8), lambda i: (0, 0)),
            pl.BlockSpec((H, 8), lambda i: (0, 0)),
        ],
        out_specs=[
            pl.BlockSpec((BLK, H), lambda i: (i, 0)),
            pl.BlockSpec((BLK, 8), lambda i: (i, 0)),
        ],
        out_shape=[
            jax.ShapeDtypeStruct((N, H), jnp.float32),
            jax.ShapeDtypeStruct((N, 8), jnp.float32),
        ],
    )(x_src, x_dst, Wsrc, Wdst, As, Ad)


def _post_body(acc_ref, hs_ref, att_ref, xd_ref, wres_ref, b_ref, rep_ref,
               g_ref, be_ref, *, heads, act):
    a_s = att_ref[:, 0:4]
    a_d = att_ref[:, 4:8]
    s = a_s + a_d
    exii = jnp.exp(jnp.where(s > 0, s, 0.2 * s))
    exf = jnp.dot(exii[:, 0:4], rep_ref[...],
                  preferred_element_type=jnp.float32)
    num = acc_ref[:, 0:H]
    den = jnp.dot(acc_ref[:, H:H + 4], rep_ref[...],
                  preferred_element_type=jnp.float32)
    num = num + exf * hs_ref[...]
    den = den + exf
    o = num / (den + 1e-16)
    o = o + jnp.dot(xd_ref[...], wres_ref[...],
                    preferred_element_type=jnp.float32) + b_ref[...]
    if act == "elu_bn":
        o = g_ref[...] * o + be_ref[...]
        o = jnp.where(o > 0, o, jnp.exp(jnp.minimum(o, 0.0)) - 1.0)
    elif act == "relu_bn":
        o = g_ref[...] * o + be_ref[...]
        o = jnp.maximum(o, 0.0)
    return o


def _post_wrap(acc_ref, hs_ref, att_ref, xd_ref, wres_ref, b_ref, rep_ref,
               g_ref, be_ref, out_ref, *, heads, act):
    out_ref[...] = _post_body(acc_ref, hs_ref, att_ref, xd_ref, wres_ref,
                              b_ref, rep_ref, g_ref, be_ref,
                              heads=heads, act=act)


def _gat_post(acc, hs, att, x_dst, Wres, b, rep, gamma, beta, heads, act):
    N = hs.shape[0]
    return pl.pallas_call(
        functools.partial(_post_wrap, heads=heads, act=act),
        grid=(N // BLK,),
        in_specs=[
            pl.BlockSpec((BLK, 72), lambda i: (i, 0)),
            pl.BlockSpec((BLK, H), lambda i: (i, 0)),
            pl.BlockSpec((BLK, 8), lambda i: (i, 0)),
            pl.BlockSpec((BLK, H), lambda i: (i, 0)),
            pl.BlockSpec((H, H), lambda i: (0, 0)),
            pl.BlockSpec((1, H), lambda i: (0, 0)),
            pl.BlockSpec((4, H), lambda i: (0, 0)),
            pl.BlockSpec((1, H), lambda i: (0, 0)),
            pl.BlockSpec((1, H), lambda i: (0, 0)),
        ],
        out_specs=pl.BlockSpec((BLK, H), lambda i: (i, 0)),
        out_shape=jax.ShapeDtypeStruct((N, H), jnp.float32),
    )(acc, hs, att, x_dst, Wres, b, rep, gamma, beta)


def _edge_phase(src, dst, hs, att, heads):
    ch = H // heads
    a_s = att[src, 0:heads]
    a_d = att[dst, 4:4 + heads]
    s = a_s + a_d
    ex = jnp.exp(jnp.where(s > 0, s, 0.2 * s))
    exf = jnp.repeat(ex, ch, axis=1)
    num = jax.ops.segment_sum(exf * hs[src], dst, num_segments=N_NODES)
    den = jax.ops.segment_sum(ex, dst, num_segments=N_NODES)
    acc = jnp.concatenate([num, den], axis=1)
    return jnp.pad(acc, ((0, 0), (0, 72 - H - heads)))


def _rep_matrix(heads):
    ch = H // heads
    rep = jnp.zeros((4, H), jnp.float32)
    for j in range(heads):
        rep = rep.at[j, j * ch:(j + 1) * ch].set(1.0)
    return rep


def _att_comb(att_s, att_d, heads, ch):
    As = jnp.zeros((H, 8), jnp.float32)
    Ad = jnp.zeros((H, 8), jnp.float32)
    for j in range(heads):
        As = As.at[j * ch:(j + 1) * ch, j].set(att_s[j])
        Ad = Ad.at[j * ch:(j + 1) * ch, 4 + j].set(att_d[j])
    return As, Ad


def kernel(agent_hist, lane_nodes, edge_index_aa, edge_index_al, params):
    p = params
    f32 = jnp.float32

    def prep_lstm(x, Din):
        T = x.shape[1]
        xt = jnp.transpose(x, (1, 0, 2))
        return jnp.pad(xt, ((0, 0), (0, 0), (0, 8 - Din)))

    ah = prep_lstm(agent_hist, 5)
    ln = prep_lstm(lane_nodes, 2)
    aw_ih = jnp.pad(p['agent_Wih'].T, ((0, 3), (0, 0)))
    lw_ih = jnp.pad(p['lane_Wih'].T, ((0, 6), (0, 0)))
    ab = (p['agent_bih'] + p['agent_bhh'])[None, :]
    lb = (p['lane_bih'] + p['lane_bhh'])[None, :]
    agent_emb = _lstm(ah, aw_ih, p['agent_Whh'].T, ab, mean=False)
    lane_emb = _lstm(ln, lw_ih, p['lane_Whh'].T, lb, mean=True)

    src_aa = edge_index_aa[0].astype(jnp.int32)
    dst_aa = edge_index_aa[1].astype(jnp.int32)
    src_al = edge_index_al[1].astype(jnp.int32)
    dst_al = edge_index_al[0].astype(jnp.int32)

    one = jnp.ones((1, H), f32)
    zero = jnp.zeros((1, H), f32)
    bn_g_aa = (p['aa_bn_gamma'] / jnp.sqrt(1.0 + 1e-5))[None, :]
    bn_b_aa = p['aa_bn_beta'][None, :]
    bn_g_al = (p['al_bn_gamma'] / jnp.sqrt(1.0 + 1e-5))[None, :]
    bn_b_al = p['al_bn_beta'][None, :]

    def gat_same(x, W, att_s, att_d, Wres, b, heads, src, dst, gamma, beta, act):
        ch = H // heads
        As, Ad = _att_comb(att_s, att_d, heads, ch)
        hs, att = _gat_pre(x, W, As + Ad)
        acc = _edge_phase(src, dst, hs, att, heads)
        return _gat_post(acc, hs, att, x, Wres, b[None, :], _rep_matrix(heads),
                         gamma, beta, heads, act)

    x = gat_same(agent_emb, p['aa0_W'], p['aa0_att_src'], p['aa0_att_dst'],
                 p['aa0_Wres'], p['aa0_b'], 4, src_aa, dst_aa,
                 bn_g_aa, bn_b_aa, "elu_bn")
    agent_social = gat_same(x, p['aa1_W'], p['aa1_att_src'], p['aa1_att_dst'],
                            p['aa1_Wres'], p['aa1_b'], 4, src_aa, dst_aa,
                            one, zero, "none")
    As, Ad = _att_comb(p['al0_att_src'], p['al0_att_dst'], 2, 32)
    hs, att = _gat_pre2(lane_emb, agent_social, p['al0_Wsrc'], p['al0_Wdst'],
                        As, Ad)
    acc = _edge_phase(src_al, dst_al, hs, att, 2)
    y = _gat_post(acc, hs, att, agent_social, p['al0_Wres'],
                  p['al0_b'][None, :], _rep_matrix(2),
                  bn_g_al, bn_b_al, 2, "relu_bn")
    agent_map = gat_same(y, p['al1_W'], p['al1_att_src'], p['al1_att_dst'],
                         p['al1_Wres'], p['al1_b'], 2, src_al, dst_al,
                         one, zero, "none")

    return (agent_emb, agent_social, agent_map, lane_emb)


# speedup vs baseline: 22.1572x; 16.0301x over previous
import functools

import jax
import jax.numpy as jnp
from jax import lax
from jax.experimental import pallas as pl
from jax.experimental.pallas import tpu as pltpu
from jax.experimental.pallas import tpu_sc as plsc

H = 64
N_NODES = 50000
BLK = 2000

E_EDGES = 800000
BATCH = 64
NSUB = 16
NBATCH = 782
EPT = BATCH * NBATCH
E_PAD = EPT * NSUB
N_PAD = 50008
HALF0 = 26000
ACC_ROWS = 26112
RPT = ACC_ROWS // NSUB
TRASH = ACC_ROWS - 1
ROWW = 72


def _lstm_body(x_ref, wih_ref, whh_ref, b_ref, out_ref, *, T, mean):
    B = x_ref.shape[1]
    h0 = jnp.zeros((B, H), jnp.float32)

    def step(t, carry):
        h, c, acc = carry
        xt = x_ref[t]
        gates = jnp.dot(xt, wih_ref[...], preferred_element_type=jnp.float32)
        gates += jnp.dot(h, whh_ref[...], preferred_element_type=jnp.float32)
        gates += b_ref[...]
        i = jax.nn.sigmoid(gates[:, 0:H])
        f = jax.nn.sigmoid(gates[:, H:2 * H])
        g = jnp.tanh(gates[:, 2 * H:3 * H])
        o = jax.nn.sigmoid(gates[:, 3 * H:4 * H])
        c = f * c + i * g
        h = o * jnp.tanh(c)
        return h, c, acc + h

    h, c, acc = lax.fori_loop(0, T, step, (h0, h0, h0))
    out_ref[...] = (acc * (1.0 / T)) if mean else h


def _lstm(x, Wih, Whh, bias, mean):
    T, N, Dp = x.shape
    grid = (N // BLK,)
    return pl.pallas_call(
        functools.partial(_lstm_body, T=T, mean=mean),
        grid=grid,
        in_specs=[
            pl.BlockSpec((T, BLK, Dp), lambda i: (0, i, 0)),
            pl.BlockSpec((Dp, 4 * H), lambda i: (0, 0)),
            pl.BlockSpec((H, 4 * H), lambda i: (0, 0)),
            pl.BlockSpec((1, 4 * H), lambda i: (0, 0)),
        ],
        out_specs=pl.BlockSpec((BLK, H), lambda i: (i, 0)),
        out_shape=jax.ShapeDtypeStruct((N, H), jnp.float32),
    )(x, Wih, Whh, bias)


def _pre_body(x_ref, w_ref, a_ref, hs_ref, att_ref):
    hs = jnp.dot(x_ref[...], w_ref[...], preferred_element_type=jnp.float32)
    hs_ref[...] = hs
    att_ref[...] = jnp.dot(hs, a_ref[...], preferred_element_type=jnp.float32)


def _gat_pre(x, W, Acomb):
    N = x.shape[0]
    return pl.pallas_call(
        _pre_body,
        grid=(N // BLK,),
        in_specs=[
            pl.BlockSpec((BLK, H), lambda i: (i, 0)),
            pl.BlockSpec((H, H), lambda i: (0, 0)),
            pl.BlockSpec((H, 8), lambda i: (0, 0)),
        ],
        out_specs=[
            pl.BlockSpec((BLK, H), lambda i: (i, 0)),
            pl.BlockSpec((BLK, 8), lambda i: (i, 0)),
        ],
        out_shape=[
            jax.ShapeDtypeStruct((N, H), jnp.float32),
            jax.ShapeDtypeStruct((N, 8), jnp.float32),
        ],
    )(x, W, Acomb)


def _pre2_body(xs_ref, xd_ref, ws_ref, wd_ref, as_ref, ad_ref, hs_ref, att_ref):
    hs = jnp.dot(xs_ref[...], ws_ref[...], preferred_element_type=jnp.float32)
    hd = jnp.dot(xd_ref[...], wd_ref[...], preferred_element_type=jnp.float32)
    hs_ref[...] = hs
    att_ref[...] = (
        jnp.dot(hs, as_ref[...], preferred_element_type=jnp.float32)
        + jnp.dot(hd, ad_ref[...], preferred_element_type=jnp.float32))


def _gat_pre2(x_src, x_dst, Wsrc, Wdst, As, Ad):
    N = x_src.shape[0]
    return pl.pallas_call(
        _pre2_body,
        grid=(N // BLK,),
        in_specs=[
            pl.BlockSpec((BLK, H), lambda i: (i, 0)),
            pl.BlockSpec((BLK, H), lambda i: (i, 0)),
            pl.BlockSpec((H, H), lambda i: (0, 0)),
            pl.BlockSpec((H, H), lambda i: (0, 0)),
            pl.BlockSpec((H, 8), lambda i: (0, 0)),
            pl.BlockSpec((H, 8), lambda i: (0, 0)),
        ],
        out_specs=[
            pl.BlockSpec((BLK, H), lambda i: (i, 0)),
            pl.BlockSpec((BLK, 8), lambda i: (i, 0)),
        ],
        out_shape=[
            jax.ShapeDtypeStruct((N, H), jnp.float32),
            jax.ShapeDtypeStruct((N, 8), jnp.float32),
        ],
    )(x_src, x_dst, Wsrc, Wdst, As, Ad)


def _post_body(acc_ref, hs_ref, att_ref, xd_ref, wres_ref, b_ref, rep_ref,
               g_ref, be_ref, *, heads, act):
    a_s = att_ref[:, 0:4]
    a_d = att_ref[:, 4:8]
    s = a_s + a_d
    exii = jnp.exp(jnp.where(s > 0, s, 0.2 * s))
    exf = jnp.dot(exii[:, 0:4], rep_ref[...],
                  preferred_element_type=jnp.float32)
    num = acc_ref[:, 0:H]
    den = jnp.dot(acc_ref[:, H:H + 4], rep_ref[...],
                  preferred_element_type=jnp.float32)
    num = num + exf * hs_ref[...]
    den = den + exf
    o = num / (den + 1e-16)
    o = o + jnp.dot(xd_ref[...], wres_ref[...],
                    preferred_element_type=jnp.float32) + b_ref[...]
    if act == "elu_bn":
        o = g_ref[...] * o + be_ref[...]
        o = jnp.where(o > 0, o, jnp.exp(jnp.minimum(o, 0.0)) - 1.0)
    elif act == "relu_bn":
        o = g_ref[...] * o + be_ref[...]
        o = jnp.maximum(o, 0.0)
    return o


def _post_wrap(acc_ref, hs_ref, att_ref, xd_ref, wres_ref, b_ref, rep_ref,
               g_ref, be_ref, out_ref, *, heads, act):
    out_ref[...] = _post_body(acc_ref, hs_ref, att_ref, xd_ref, wres_ref,
                              b_ref, rep_ref, g_ref, be_ref,
                              heads=heads, act=act)


def _gat_post(acc, hs, att, x_dst, Wres, b, rep, gamma, beta, heads, act,
              nblocks, row_off):
    return pl.pallas_call(
        functools.partial(_post_wrap, heads=heads, act=act),
        grid=(nblocks,),
        in_specs=[
            pl.BlockSpec((BLK, ROWW), lambda i: (i, 0)),
            pl.BlockSpec((BLK, H), lambda i: (i + row_off, 0)),
            pl.BlockSpec((BLK, 8), lambda i: (i + row_off, 0)),
            pl.BlockSpec((BLK, H), lambda i: (i + row_off, 0)),
            pl.BlockSpec((H, H), lambda i: (0, 0)),
            pl.BlockSpec((1, H), lambda i: (0, 0)),
            pl.BlockSpec((4, H), lambda i: (0, 0)),
            pl.BlockSpec((1, H), lambda i: (0, 0)),
            pl.BlockSpec((1, H), lambda i: (0, 0)),
        ],
        out_specs=pl.BlockSpec((BLK, H), lambda i: (i, 0)),
        out_shape=jax.ShapeDtypeStruct((nblocks * BLK, H), jnp.float32),
    )(acc, hs, att, x_dst, Wres, b, rep, gamma, beta)


def _edge_body(src_hbm, dst_hbm, att_hbm, hs_hbm, zeros_hbm, out_hbm,
               srcv, dstv, rowv, atts, attd, hsr, msg,
               acc, sem1, sem2, sem3, *, heads):
    ch = H // heads
    c = lax.axis_index("c")
    s = lax.axis_index("s")
    off = c * HALF0
    bound = HALF0 - 2000 * c

    pltpu.sync_copy(zeros_hbm.at[pl.ds(s * RPT, RPT)],
                    acc.at[pl.ds(s * RPT, RPT)])
    plsc.subcore_barrier()

    iota = lax.iota(jnp.int32, 16)
    ebase = s * EPT

    def batch(b, carry):
        base = ebase + b * BATCH
        pltpu.sync_copy(src_hbm.at[pl.ds(base, BATCH)], srcv)
        pltpu.sync_copy(dst_hbm.at[pl.ds(base, BATCH)], dstv)
        cp1 = pltpu.async_copy(att_hbm.at[srcv], atts, sem1)
        cp2 = pltpu.async_copy(att_hbm.at[dstv], attd, sem2)
        cp3 = pltpu.async_copy(hs_hbm.at[srcv], hsr, sem3)
        cp1.wait()
        cp2.wait()
        cp3.wait()
        for g in range(BATCH // 16):
            ev = iota + (g * 16)
            vd = dstv[pl.ds(g * 16, 16)]
            dl = vd - off
            okm = (dl >= 0) & (dl < bound)
            rowv[pl.ds(g * 16, 16)] = jnp.where(okm, dl, TRASH)
            for j in range(heads):
                va = plsc.load_gather(atts, [ev, jnp.full((16,), j, jnp.int32)])
                vb = plsc.load_gather(attd, [ev, jnp.full((16,), 4 + j, jnp.int32)])
                sv = va + vb
                ex = jnp.exp(jnp.where(sv > 0.0, sv, 0.2 * sv))
                plsc.store_scatter(msg, [ev, jnp.full((16,), H + j, jnp.int32)], ex)
                for t in range(ch):
                    f = j * ch + t
                    fv = jnp.full((16,), f, jnp.int32)
                    vh = plsc.load_gather(hsr, [ev, fv])
                    plsc.store_scatter(msg, [ev, fv], vh * ex)
        pltpu.sync_copy(msg, acc.at[rowv], add=True)
        return carry

    for g in range(BATCH // 16):
        ev = iota + (g * 16)
        for jpad in range(H + heads, ROWW):
            plsc.store_scatter(msg, [ev, jnp.full((16,), jpad, jnp.int32)],
                               jnp.zeros((16,), jnp.float32))
    lax.fori_loop(0, NBATCH, batch, 0)
    plsc.subcore_barrier()
    pltpu.sync_copy(acc.at[pl.ds(s * RPT, RPT)],
                    out_hbm.at[c, pl.ds(s * RPT, RPT)])


def _edge_sc(src_p, dst_p, att_p, hs_p, zeros, heads):
    mesh = plsc.VectorSubcoreMesh(core_axis_name="c", subcore_axis_name="s",
                                  num_cores=2, num_subcores=NSUB)
    kern = pl.kernel(
        functools.partial(_edge_body, heads=heads),
        out_type=jax.ShapeDtypeStruct((2, ACC_ROWS, ROWW), jnp.float32),
        mesh=mesh,
        compiler_params=pltpu.CompilerParams(needs_layout_passes=False,
                                             use_tc_tiling_on_sc=False,
                                             internal_scratch_in_bytes=512 * 1024),
        scratch_types=[
            pltpu.VMEM((BATCH,), jnp.int32),
            pltpu.VMEM((BATCH,), jnp.int32),
            pltpu.VMEM((BATCH,), jnp.int32),
            pltpu.VMEM((BATCH, 8), jnp.float32),
            pltpu.VMEM((BATCH, 8), jnp.float32),
            pltpu.VMEM((BATCH, H), jnp.float32),
            pltpu.VMEM((BATCH, ROWW), jnp.float32),
            pltpu.VMEM_SHARED((ACC_ROWS, ROWW), jnp.float32),
            pltpu.SemaphoreType.DMA,
            pltpu.SemaphoreType.DMA,
            pltpu.SemaphoreType.DMA,
        ],
    )
    return kern(src_p, dst_p, att_p, hs_p, zeros)


def _rep_matrix(heads):
    ch = H // heads
    rep = jnp.zeros((4, H), jnp.float32)
    for j in range(heads):
        rep = rep.at[j, j * ch:(j + 1) * ch].set(1.0)
    return rep


def _att_comb(att_s, att_d, heads, ch):
    As = jnp.zeros((H, 8), jnp.float32)
    Ad = jnp.zeros((H, 8), jnp.float32)
    for j in range(heads):
        As = As.at[j * ch:(j + 1) * ch, j].set(att_s[j])
        Ad = Ad.at[j * ch:(j + 1) * ch, 4 + j].set(att_d[j])
    return As, Ad


def kernel(agent_hist, lane_nodes, edge_index_aa, edge_index_al, params):
    p = params
    f32 = jnp.float32

    def prep_lstm(x, Din):
        T = x.shape[1]
        xt = jnp.transpose(x, (1, 0, 2))
        return jnp.pad(xt, ((0, 0), (0, 0), (0, 8 - Din)))

    ah = prep_lstm(agent_hist, 5)
    ln = prep_lstm(lane_nodes, 2)
    aw_ih = jnp.pad(p['agent_Wih'].T, ((0, 3), (0, 0)))
    lw_ih = jnp.pad(p['lane_Wih'].T, ((0, 6), (0, 0)))
    ab = (p['agent_bih'] + p['agent_bhh'])[None, :]
    lb = (p['lane_bih'] + p['lane_bhh'])[None, :]
    agent_emb = _lstm(ah, aw_ih, p['agent_Whh'].T, ab, mean=False)
    lane_emb = _lstm(ln, lw_ih, p['lane_Whh'].T, lb, mean=True)

    epad = jnp.full((E_PAD - E_EDGES,), N_NODES, jnp.int32)
    src_aa = jnp.concatenate([edge_index_aa[0].astype(jnp.int32), epad])
    dst_aa = jnp.concatenate([edge_index_aa[1].astype(jnp.int32), epad])
    src_al = jnp.concatenate([edge_index_al[1].astype(jnp.int32), epad])
    dst_al = jnp.concatenate([edge_index_al[0].astype(jnp.int32), epad])
    zeros_acc = jnp.zeros((ACC_ROWS, ROWW), f32)

    def pad_tab(a):
        return jnp.pad(a, ((0, N_PAD - N_NODES), (0, 0)))

    one = jnp.ones((1, H), f32)
    zero = jnp.zeros((1, H), f32)
    bn_g_aa = (p['aa_bn_gamma'] / jnp.sqrt(1.0 + 1e-5))[None, :]
    bn_b_aa = p['aa_bn_beta'][None, :]
    bn_g_al = (p['al_bn_gamma'] / jnp.sqrt(1.0 + 1e-5))[None, :]
    bn_b_al = p['al_bn_beta'][None, :]

    NB0 = HALF0 // BLK
    NB1 = (N_NODES - HALF0) // BLK

    def post_both(accs, hs, att, x_dst, Wres, b, heads, gamma, beta, act):
        rep = _rep_matrix(heads)
        o0 = _gat_post(accs[0], hs, att, x_dst, Wres, b[None, :], rep,
                       gamma, beta, heads, act, NB0, 0)
        o1 = _gat_post(accs[1], hs, att, x_dst, Wres, b[None, :], rep,
                       gamma, beta, heads, act, NB1, NB0)
        return jnp.concatenate([o0, o1], axis=0)

    def gat_same(x, W, att_s, att_d, Wres, b, heads, src, dst, gamma, beta, act):
        ch = H // heads
        As, Ad = _att_comb(att_s, att_d, heads, ch)
        hs, att = _gat_pre(x, W, As + Ad)
        accs = _edge_sc(src, dst, pad_tab(att), pad_tab(hs), zeros_acc, heads)
        return post_both(accs, hs, att, x, Wres, b, heads, gamma, beta, act)

    x = gat_same(agent_emb, p['aa0_W'], p['aa0_att_src'], p['aa0_att_dst'],
                 p['aa0_Wres'], p['aa0_b'], 4, src_aa, dst_aa,
                 bn_g_aa, bn_b_aa, "elu_bn")
    agent_social = gat_same(x, p['aa1_W'], p['aa1_att_src'], p['aa1_att_dst'],
                            p['aa1_Wres'], p['aa1_b'], 4, src_aa, dst_aa,
                            one, zero, "none")
    As, Ad = _att_comb(p['al0_att_src'], p['al0_att_dst'], 2, 32)
    hs, att = _gat_pre2(lane_emb, agent_social, p['al0_Wsrc'], p['al0_Wdst'],
                        As, Ad)
    accs = _edge_sc(src_al, dst_al, pad_tab(att), pad_tab(hs), zeros_acc, 2)
    y = post_both(accs, hs, att, agent_social, p['al0_Wres'], p['al0_b'],
                  2, bn_g_al, bn_b_al, "relu_bn")
    agent_map = gat_same(y, p['al1_W'], p['al1_att_src'], p['al1_att_dst'],
                         p['al1_Wres'], p['al1_b'], 2, src_al, dst_al,
                         one, zero, "none")

    return (agent_emb, agent_social, agent_map, lane_emb)


# speedup vs baseline: 22.5354x; 1.0171x over previous
import functools

import jax
import jax.numpy as jnp
from jax import lax
from jax.experimental import pallas as pl
from jax.experimental.pallas import tpu as pltpu
from jax.experimental.pallas import tpu_sc as plsc

H = 64
N_NODES = 50000
BLK = 2000

E_EDGES = 800000
BATCH = 32
NSLOT = 3
NSUB = 16
NBATCH = 1566
EPT = BATCH * NBATCH
E_PAD = EPT * NSUB
E_ALLOC = E_PAD
N_PAD = 50008
HALF = 25000
ACC_ROWS = 25088
RPT = ACC_ROWS // NSUB
TRASH = ACC_ROWS - 1
ROWW = 72
PBLK = 1000


def _lstm_body(x_ref, wih_ref, whh_ref, b_ref, out_ref, *, T, mean):
    B = x_ref.shape[1]
    h0 = jnp.zeros((B, H), jnp.float32)

    def step(t, carry):
        h, c, acc = carry
        xt = x_ref[t]
        gates = jnp.dot(xt, wih_ref[...], preferred_element_type=jnp.float32)
        gates += jnp.dot(h, whh_ref[...], preferred_element_type=jnp.float32)
        gates += b_ref[...]
        i = jax.nn.sigmoid(gates[:, 0:H])
        f = jax.nn.sigmoid(gates[:, H:2 * H])
        g = jnp.tanh(gates[:, 2 * H:3 * H])
        o = jax.nn.sigmoid(gates[:, 3 * H:4 * H])
        c = f * c + i * g
        h = o * jnp.tanh(c)
        return h, c, acc + h

    h, c, acc = lax.fori_loop(0, T, step, (h0, h0, h0))
    out_ref[...] = (acc * (1.0 / T)) if mean else h


def _lstm(x, Wih, Whh, bias, mean):
    T, N, Dp = x.shape
    grid = (N // BLK,)
    return pl.pallas_call(
        functools.partial(_lstm_body, T=T, mean=mean),
        grid=grid,
        in_specs=[
            pl.BlockSpec((T, BLK, Dp), lambda i: (0, i, 0)),
            pl.BlockSpec((Dp, 4 * H), lambda i: (0, 0)),
            pl.BlockSpec((H, 4 * H), lambda i: (0, 0)),
            pl.BlockSpec((1, 4 * H), lambda i: (0, 0)),
        ],
        out_specs=pl.BlockSpec((BLK, H), lambda i: (i, 0)),
        out_shape=jax.ShapeDtypeStruct((N, H), jnp.float32),
    )(x, Wih, Whh, bias)


def _pre_body(x_ref, w_ref, a_ref, hs_ref, att_ref):
    hs = jnp.dot(x_ref[...], w_ref[...], preferred_element_type=jnp.float32)
    hs_ref[...] = hs
    att_ref[...] = jnp.dot(hs, a_ref[...], preferred_element_type=jnp.float32)


def _gat_pre(x, W, Acomb):
    N = x.shape[0]
    return pl.pallas_call(
        _pre_body,
        grid=(N // BLK,),
        in_specs=[
            pl.BlockSpec((BLK, H), lambda i: (i, 0)),
            pl.BlockSpec((H, H), lambda i: (0, 0)),
            pl.BlockSpec((H, 8), lambda i: (0, 0)),
        ],
        out_specs=[
            pl.BlockSpec((BLK, H), lambda i: (i, 0)),
            pl.BlockSpec((BLK, 8), lambda i: (i, 0)),
        ],
        out_shape=[
            jax.ShapeDtypeStruct((N, H), jnp.float32),
            jax.ShapeDtypeStruct((N, 8), jnp.float32),
        ],
    )(x, W, Acomb)


def _pre2_body(xs_ref, xd_ref, ws_ref, wd_ref, as_ref, ad_ref, hs_ref, att_ref):
    hs = jnp.dot(xs_ref[...], ws_ref[...], preferred_element_type=jnp.float32)
    hd = jnp.dot(xd_ref[...], wd_ref[...], preferred_element_type=jnp.float32)
    hs_ref[...] = hs
    att_ref[...] = (
        jnp.dot(hs, as_ref[...], preferred_element_type=jnp.float32)
        + jnp.dot(hd, ad_ref[...], preferred_element_type=jnp.float32))


def _gat_pre2(x_src, x_dst, Wsrc, Wdst, As, Ad):
    N = x_src.shape[0]
    return pl.pallas_call(
        _pre2_body,
        grid=(N // BLK,),
        in_specs=[
            pl.BlockSpec((BLK, H), lambda i: (i, 0)),
            pl.BlockSpec((BLK, H), lambda i: (i, 0)),
            pl.BlockSpec((H, H), lambda i: (0, 0)),
            pl.BlockSpec((H, H), lambda i: (0, 0)),
            pl.BlockSpec((H, 8), lambda i: (0, 0)),
            pl.BlockSpec((H, 8), lambda i: (0, 0)),
        ],
        out_specs=[
            pl.BlockSpec((BLK, H), lambda i: (i, 0)),
            pl.BlockSpec((BLK, 8), lambda i: (i, 0)),
        ],
        out_shape=[
            jax.ShapeDtypeStruct((N, H), jnp.float32),
            jax.ShapeDtypeStruct((N, 8), jnp.float32),
        ],
    )(x_src, x_dst, Wsrc, Wdst, As, Ad)


def _post_body(acc_ref, hs_ref, att_ref, xd_ref, wres_ref, b_ref, rep_ref,
               g_ref, be_ref, *, heads, act):
    a_s = att_ref[:, 0:4]
    a_d = att_ref[:, 4:8]
    s = a_s + a_d
    exii = jnp.exp(jnp.where(s > 0, s, 0.2 * s))
    exf = jnp.dot(exii[:, 0:4], rep_ref[...],
                  preferred_element_type=jnp.float32)
    num = acc_ref[:, 0:H]
    den = jnp.dot(acc_ref[:, H:H + 4], rep_ref[...],
                  preferred_element_type=jnp.float32)
    num = num + exf * hs_ref[...]
    den = den + exf
    o = num / (den + 1e-16)
    o = o + jnp.dot(xd_ref[...], wres_ref[...],
                    preferred_element_type=jnp.float32) + b_ref[...]
    if act == "elu_bn":
        o = g_ref[...] * o + be_ref[...]
        o = jnp.where(o > 0, o, jnp.exp(jnp.minimum(o, 0.0)) - 1.0)
    elif act == "relu_bn":
        o = g_ref[...] * o + be_ref[...]
        o = jnp.maximum(o, 0.0)
    return o


def _post_wrap(acc_ref, hs_ref, att_ref, xd_ref, wres_ref, b_ref, rep_ref,
               g_ref, be_ref, out_ref, *, heads, act):
    out_ref[...] = _post_body(acc_ref, hs_ref, att_ref, xd_ref, wres_ref,
                              b_ref, rep_ref, g_ref, be_ref,
                              heads=heads, act=act)


def _gat_post(acc, hs, att, x_dst, Wres, b, rep, gamma, beta, heads, act,
              nblocks, row_off):
    return pl.pallas_call(
        functools.partial(_post_wrap, heads=heads, act=act),
        grid=(nblocks,),
        in_specs=[
            pl.BlockSpec((PBLK, ROWW), lambda i: (i, 0)),
            pl.BlockSpec((PBLK, H), lambda i: (i + row_off, 0)),
            pl.BlockSpec((PBLK, 8), lambda i: (i + row_off, 0)),
            pl.BlockSpec((PBLK, H), lambda i: (i + row_off, 0)),
            pl.BlockSpec((H, H), lambda i: (0, 0)),
            pl.BlockSpec((1, H), lambda i: (0, 0)),
            pl.BlockSpec((4, H), lambda i: (0, 0)),
            pl.BlockSpec((1, H), lambda i: (0, 0)),
            pl.BlockSpec((1, H), lambda i: (0, 0)),
        ],
        out_specs=pl.BlockSpec((PBLK, H), lambda i: (i, 0)),
        out_shape=jax.ShapeDtypeStruct((nblocks * PBLK, H), jnp.float32),
    )(acc, hs, att, x_dst, Wres, b, rep, gamma, beta)


def _edge_body(src_hbm, dst_hbm, att_hbm, hs_hbm, zeros_hbm, out_hbm,
               srcv0, srcv1, srcv2, dstv0, dstv1, dstv2,
               rowv0, rowv1, rowv2, atts0, atts1, atts2,
               attd0, attd1, attd2, hsr0, hsr1, hsr2, msg0, msg1, msg2,
               acc, semg, sems, *, heads):
    ch = H // heads
    c = lax.axis_index("c")
    s = lax.axis_index("s")
    off = c * HALF
    srcv = (srcv0, srcv1, srcv2)
    dstv = (dstv0, dstv1, dstv2)
    rowv = (rowv0, rowv1, rowv2)
    atts = (atts0, atts1, atts2)
    attd = (attd0, attd1, attd2)
    hsr = (hsr0, hsr1, hsr2)
    msg = (msg0, msg1, msg2)

    pltpu.sync_copy(zeros_hbm.at[pl.ds(s * RPT, RPT)],
                    acc.at[pl.ds(s * RPT, RPT)])
    plsc.subcore_barrier()

    iota = lax.iota(jnp.int32, 16)
    ebase = s * EPT

    def compute(slot):
        def group(g, carry):
            g16 = pl.multiple_of(g * 16, 16)
            ev = iota + g16
            vd = dstv[slot][pl.ds(g16, 16)]
            dl = vd - off
            okm = (dl >= 0) & (dl < HALF)
            rowv[slot][pl.ds(g16, 16)] = jnp.where(okm, dl, TRASH)
            for j in range(heads):
                va = plsc.load_gather(atts[slot], [ev, jnp.full((16,), j, jnp.int32)])
                vb = plsc.load_gather(attd[slot], [ev, jnp.full((16,), 4 + j, jnp.int32)])
                sv = va + vb
                ex = jnp.exp(jnp.where(sv > 0.0, sv, 0.2 * sv))
                plsc.store_scatter(msg[slot], [ev, jnp.full((16,), H + j, jnp.int32)], ex)
                for t in range(ch):
                    f = j * ch + t
                    fv = jnp.full((16,), f, jnp.int32)
                    vh = plsc.load_gather(hsr[slot], [ev, fv])
                    plsc.store_scatter(msg[slot], [ev, fv], vh * ex)
            return carry

        lax.fori_loop(0, BATCH // 16, group, 0)

    for slot in range(NSLOT):
        for g in range(BATCH // 16):
            ev = iota + (g * 16)
            for jpad in range(H + heads, ROWW):
                plsc.store_scatter(msg[slot],
                                   [ev, jnp.full((16,), jpad, jnp.int32)],
                                   jnp.zeros((16,), jnp.float32))

    def trio(p, carry):
        bb = p * NSLOT
        gath = []
        for slot in range(NSLOT):
            base = ebase + (bb + slot) * BATCH
            pltpu.sync_copy(src_hbm.at[pl.ds(base, BATCH)], srcv[slot])
            pltpu.sync_copy(dst_hbm.at[pl.ds(base, BATCH)], dstv[slot])
            gath.append((
                pltpu.async_copy(att_hbm.at[srcv[slot]], atts[slot], semg),
                pltpu.async_copy(att_hbm.at[dstv[slot]], attd[slot], semg),
                pltpu.async_copy(hs_hbm.at[srcv[slot]], hsr[slot], semg),
            ))
        scat = []
        for slot in range(NSLOT):
            for d in gath[slot]:
                d.wait()
            compute(slot)
            scat.append(pltpu.async_copy(msg[slot], acc.at[rowv[slot]],
                                         sems, add=True))
        for d in scat:
            d.wait()
        return carry

    lax.fori_loop(0, NBATCH // NSLOT, trio, 0)

    plsc.subcore_barrier()
    pltpu.sync_copy(acc.at[pl.ds(s * RPT, RPT)],
                    out_hbm.at[c, pl.ds(s * RPT, RPT)])


def _edge_sc(src_p, dst_p, att_p, hs_p, zeros, heads):
    mesh = plsc.VectorSubcoreMesh(core_axis_name="c", subcore_axis_name="s",
                                  num_cores=2, num_subcores=NSUB)
    kern = pl.kernel(
        functools.partial(_edge_body, heads=heads),
        out_type=jax.ShapeDtypeStruct((2, ACC_ROWS, ROWW), jnp.float32),
        mesh=mesh,
        compiler_params=pltpu.CompilerParams(needs_layout_passes=False,
                                             use_tc_tiling_on_sc=False,
                                             internal_scratch_in_bytes=512 * 1024),
        scratch_types=(
            [pltpu.VMEM((BATCH,), jnp.int32)] * (3 * NSLOT)
            + [pltpu.VMEM((BATCH, 8), jnp.float32)] * (2 * NSLOT)
            + [pltpu.VMEM((BATCH, H), jnp.float32)] * NSLOT
            + [pltpu.VMEM((BATCH, ROWW), jnp.float32)] * NSLOT
            + [pltpu.VMEM_SHARED((ACC_ROWS, ROWW), jnp.float32),
               pltpu.SemaphoreType.DMA, pltpu.SemaphoreType.DMA]
        ),
    )
    return kern(src_p, dst_p, att_p, hs_p, zeros)


def _rep_matrix(heads):
    ch = H // heads
    rep = jnp.zeros((4, H), jnp.float32)
    for j in range(heads):
        rep = rep.at[j, j * ch:(j + 1) * ch].set(1.0)
    return rep


def _att_comb(att_s, att_d, heads, ch):
    As = jnp.zeros((H, 8), jnp.float32)
    Ad = jnp.zeros((H, 8), jnp.float32)
    for j in range(heads):
        As = As.at[j * ch:(j + 1) * ch, j].set(att_s[j])
        Ad = Ad.at[j * ch:(j + 1) * ch, 4 + j].set(att_d[j])
    return As, Ad


def kernel(agent_hist, lane_nodes, edge_index_aa, edge_index_al, params):
    p = params
    f32 = jnp.float32

    def prep_lstm(x, Din):
        T = x.shape[1]
        xt = jnp.transpose(x, (1, 0, 2))
        return jnp.pad(xt, ((0, 0), (0, 0), (0, 8 - Din)))

    ah = prep_lstm(agent_hist, 5)
    ln = prep_lstm(lane_nodes, 2)
    aw_ih = jnp.pad(p['agent_Wih'].T, ((0, 3), (0, 0)))
    lw_ih = jnp.pad(p['lane_Wih'].T, ((0, 6), (0, 0)))
    ab = (p['agent_bih'] + p['agent_bhh'])[None, :]
    lb = (p['lane_bih'] + p['lane_bhh'])[None, :]
    agent_emb = _lstm(ah, aw_ih, p['agent_Whh'].T, ab, mean=False)
    lane_emb = _lstm(ln, lw_ih, p['lane_Whh'].T, lb, mean=True)

    epad = jnp.full((E_ALLOC - E_EDGES,), N_NODES, jnp.int32)
    src_aa = jnp.concatenate([edge_index_aa[0].astype(jnp.int32), epad])
    dst_aa = jnp.concatenate([edge_index_aa[1].astype(jnp.int32), epad])
    src_al = jnp.concatenate([edge_index_al[1].astype(jnp.int32), epad])
    dst_al = jnp.concatenate([edge_index_al[0].astype(jnp.int32), epad])
    zeros_acc = jnp.zeros((ACC_ROWS, ROWW), f32)

    def pad_tab(a):
        return jnp.pad(a, ((0, N_PAD - N_NODES), (0, 0)))

    one = jnp.ones((1, H), f32)
    zero = jnp.zeros((1, H), f32)
    bn_g_aa = (p['aa_bn_gamma'] / jnp.sqrt(1.0 + 1e-5))[None, :]
    bn_b_aa = p['aa_bn_beta'][None, :]
    bn_g_al = (p['al_bn_gamma'] / jnp.sqrt(1.0 + 1e-5))[None, :]
    bn_b_al = p['al_bn_beta'][None, :]

    NB0 = HALF // PBLK
    NB1 = (N_NODES - HALF) // PBLK

    def post_both(accs, hs, att, x_dst, Wres, b, heads, gamma, beta, act):
        rep = _rep_matrix(heads)
        o0 = _gat_post(accs[0], hs, att, x_dst, Wres, b[None, :], rep,
                       gamma, beta, heads, act, NB0, 0)
        o1 = _gat_post(accs[1], hs, att, x_dst, Wres, b[None, :], rep,
                       gamma, beta, heads, act, NB1, NB0)
        return jnp.concatenate([o0, o1], axis=0)

    def gat_same(x, W, att_s, att_d, Wres, b, heads, src, dst, gamma, beta, act):
        ch = H // heads
        As, Ad = _att_comb(att_s, att_d, heads, ch)
        hs, att = _gat_pre(x, W, As + Ad)
        accs = _edge_sc(src, dst, pad_tab(att), pad_tab(hs), zeros_acc, heads)
        return post_both(accs, hs, att, x, Wres, b, heads, gamma, beta, act)

    x = gat_same(agent_emb, p['aa0_W'], p['aa0_att_src'], p['aa0_att_dst'],
                 p['aa0_Wres'], p['aa0_b'], 4, src_aa, dst_aa,
                 bn_g_aa, bn_b_aa, "elu_bn")
    agent_social = gat_same(x, p['aa1_W'], p['aa1_att_src'], p['aa1_att_dst'],
                            p['aa1_Wres'], p['aa1_b'], 4, src_aa, dst_aa,
                            one, zero, "none")
    As, Ad = _att_comb(p['al0_att_src'], p['al0_att_dst'], 2, 32)
    hs, att = _gat_pre2(lane_emb, agent_social, p['al0_Wsrc'], p['al0_Wdst'],
                        As, Ad)
    accs = _edge_sc(src_al, dst_al, pad_tab(att), pad_tab(hs), zeros_acc, 2)
    y = post_both(accs, hs, att, agent_social, p['al0_Wres'], p['al0_b'],
                  2, bn_g_al, bn_b_al, "relu_bn")
    agent_map = gat_same(y, p['al1_W'], p['al1_att_src'], p['al1_att_dst'],
                         p['al1_Wres'], p['al1_b'], 2, src_al, dst_al,
                         one, zero, "none")

    return (agent_emb, agent_social, agent_map, lane_emb)


# speedup vs baseline: 40.0744x; 1.7783x over previous
import functools

import jax
import jax.numpy as jnp
from jax import lax
from jax.experimental import pallas as pl
from jax.experimental.pallas import tpu as pltpu
from jax.experimental.pallas import tpu_sc as plsc

H = 64
N_NODES = 50000
BLK = 2000

E_EDGES = 800000
BATCH = 96
NSUB = 16
NBATCH = 522
EPT = BATCH * NBATCH
E_PAD = EPT * NSUB
NBT = E_PAD // BATCH
N_PAD = 50008
HALF = 25000
ACC_ROWS = 25088
RPT = ACC_ROWS // NSUB
TRASH = ACC_ROWS - 1
ROWW = 72
PBLK = 1000


def _lstm_body(x_ref, wih_ref, whh_ref, b_ref, out_ref, *, T, mean):
    B = x_ref.shape[1]
    h0 = jnp.zeros((B, H), jnp.float32)

    def step(t, carry):
        h, c, acc = carry
        xt = x_ref[t]
        gates = jnp.dot(xt, wih_ref[...], preferred_element_type=jnp.float32)
        gates += jnp.dot(h, whh_ref[...], preferred_element_type=jnp.float32)
        gates += b_ref[...]
        i = jax.nn.sigmoid(gates[:, 0:H])
        f = jax.nn.sigmoid(gates[:, H:2 * H])
        g = jnp.tanh(gates[:, 2 * H:3 * H])
        o = jax.nn.sigmoid(gates[:, 3 * H:4 * H])
        c = f * c + i * g
        h = o * jnp.tanh(c)
        return h, c, acc + h

    h, c, acc = lax.fori_loop(0, T, step, (h0, h0, h0))
    out_ref[...] = (acc * (1.0 / T)) if mean else h


def _lstm(x, Wih, Whh, bias, mean):
    T, N, Dp = x.shape
    grid = (N // BLK,)
    return pl.pallas_call(
        functools.partial(_lstm_body, T=T, mean=mean),
        grid=grid,
        in_specs=[
            pl.BlockSpec((T, BLK, Dp), lambda i: (0, i, 0)),
            pl.BlockSpec((Dp, 4 * H), lambda i: (0, 0)),
            pl.BlockSpec((H, 4 * H), lambda i: (0, 0)),
            pl.BlockSpec((1, 4 * H), lambda i: (0, 0)),
        ],
        out_specs=pl.BlockSpec((BLK, H), lambda i: (i, 0)),
        out_shape=jax.ShapeDtypeStruct((N, H), jnp.float32),
    )(x, Wih, Whh, bias)


def _pre_body(x_ref, w_ref, as_ref, ad_ref, big_ref, adt_ref):
    hs = jnp.dot(x_ref[...], w_ref[...], preferred_element_type=jnp.float32)
    a_s = jnp.dot(hs, as_ref[...], preferred_element_type=jnp.float32)
    B = hs.shape[0]
    big_ref[...] = jnp.concatenate(
        [hs, a_s, jnp.zeros((B, ROWW - H - 4), jnp.float32)], axis=1)
    adt_ref[...] = jnp.dot(hs, ad_ref[...], preferred_element_type=jnp.float32)


def _gat_pre(x, W, As, Ad):
    N = x.shape[0]
    return pl.pallas_call(
        _pre_body,
        grid=(N // BLK,),
        in_specs=[
            pl.BlockSpec((BLK, H), lambda i: (i, 0)),
            pl.BlockSpec((H, H), lambda i: (0, 0)),
            pl.BlockSpec((H, 4), lambda i: (0, 0)),
            pl.BlockSpec((H, 4), lambda i: (0, 0)),
        ],
        out_specs=[
            pl.BlockSpec((BLK, ROWW), lambda i: (i, 0)),
            pl.BlockSpec((BLK, 4), lambda i: (i, 0)),
        ],
        out_shape=[
            jax.ShapeDtypeStruct((N, ROWW), jnp.float32),
            jax.ShapeDtypeStruct((N, 4), jnp.float32),
        ],
    )(x, W, As, Ad)


def _pre2_body(xs_ref, xd_ref, ws_ref, wd_ref, as_ref, ad_ref,
               big_ref, adt_ref):
    hs = jnp.dot(xs_ref[...], ws_ref[...], preferred_element_type=jnp.float32)
    hd = jnp.dot(xd_ref[...], wd_ref[...], preferred_element_type=jnp.float32)
    a_s = jnp.dot(hs, as_ref[...], preferred_element_type=jnp.float32)
    B = hs.shape[0]
    big_ref[...] = jnp.concatenate(
        [hs, a_s, jnp.zeros((B, ROWW - H - 4), jnp.float32)], axis=1)
    adt_ref[...] = jnp.dot(hd, ad_ref[...], preferred_element_type=jnp.float32)


def _gat_pre2(x_src, x_dst, Wsrc, Wdst, As, Ad):
    N = x_src.shape[0]
    return pl.pallas_call(
        _pre2_body,
        grid=(N // BLK,),
        in_specs=[
            pl.BlockSpec((BLK, H), lambda i: (i, 0)),
            pl.BlockSpec((BLK, H), lambda i: (i, 0)),
            pl.BlockSpec((H, H), lambda i: (0, 0)),
            pl.BlockSpec((H, H), lambda i: (0, 0)),
            pl.BlockSpec((H, 4), lambda i: (0, 0)),
            pl.BlockSpec((H, 4), lambda i: (0, 0)),
        ],
        out_specs=[
            pl.BlockSpec((BLK, ROWW), lambda i: (i, 0)),
            pl.BlockSpec((BLK, 4), lambda i: (i, 0)),
        ],
        out_shape=[
            jax.ShapeDtypeStruct((N, ROWW), jnp.float32),
            jax.ShapeDtypeStruct((N, 4), jnp.float32),
        ],
    )(x_src, x_dst, Wsrc, Wdst, As, Ad)


def _post_body(acc_ref, big_ref, adt_ref, xd_ref, wres_ref, b_ref, rep_ref,
               g_ref, be_ref, *, heads, act):
    hs_vals = big_ref[:, 0:H]
    a_s = big_ref[:, H:H + 4]
    a_d = adt_ref[...]
    s = a_s + a_d
    exii = jnp.exp(jnp.where(s > 0, s, 0.2 * s))
    exf = jnp.dot(exii[:, 0:4], rep_ref[...],
                  preferred_element_type=jnp.float32)
    num = acc_ref[:, 0:H]
    den = jnp.dot(acc_ref[:, H:H + 4], rep_ref[...],
                  preferred_element_type=jnp.float32)
    num = num + exf * hs_vals
    den = den + exf
    o = num / (den + 1e-16)
    o = o + jnp.dot(xd_ref[...], wres_ref[...],
                    preferred_element_type=jnp.float32) + b_ref[...]
    if act == "elu_bn":
        o = g_ref[...] * o + be_ref[...]
        o = jnp.where(o > 0, o, jnp.exp(jnp.minimum(o, 0.0)) - 1.0)
    elif act == "relu_bn":
        o = g_ref[...] * o + be_ref[...]
        o = jnp.maximum(o, 0.0)
    return o


def _post_wrap(acc_ref, big_ref, adt_ref, xd_ref, wres_ref, b_ref, rep_ref,
               g_ref, be_ref, out_ref, *, heads, act):
    out_ref[...] = _post_body(acc_ref, big_ref, adt_ref, xd_ref, wres_ref,
                              b_ref, rep_ref, g_ref, be_ref,
                              heads=heads, act=act)


def _gat_post(acc, big, adt, x_dst, Wres, b, rep, gamma, beta, heads, act,
              nblocks, row_off):
    return pl.pallas_call(
        functools.partial(_post_wrap, heads=heads, act=act),
        grid=(nblocks,),
        in_specs=[
            pl.BlockSpec((PBLK, ROWW), lambda i: (i, 0)),
            pl.BlockSpec((PBLK, ROWW), lambda i: (i + row_off, 0)),
            pl.BlockSpec((PBLK, 4), lambda i: (i + row_off, 0)),
            pl.BlockSpec((PBLK, H), lambda i: (i + row_off, 0)),
            pl.BlockSpec((H, H), lambda i: (0, 0)),
            pl.BlockSpec((1, H), lambda i: (0, 0)),
            pl.BlockSpec((4, H), lambda i: (0, 0)),
            pl.BlockSpec((1, H), lambda i: (0, 0)),
            pl.BlockSpec((1, H), lambda i: (0, 0)),
        ],
        out_specs=pl.BlockSpec((PBLK, H), lambda i: (i, 0)),
        out_shape=jax.ShapeDtypeStruct((nblocks * PBLK, H), jnp.float32),
    )(acc, big, adt, x_dst, Wres, b, rep, gamma, beta)


def _edge_body(eidx_hbm, big_hbm, ad_hbm, zeros_hbm, out_hbm,
               srcv0, srcv1, dstv0, dstv1, rowv0, rowv1,
               buf0, buf1, adb0, adb1,
               acc, semg, *, heads):
    ch = H // heads
    c = lax.axis_index("c")
    s = lax.axis_index("s")
    off = c * HALF
    srcv = (srcv0, srcv1)
    dstv = (dstv0, dstv1)
    rowv = (rowv0, rowv1)
    buf = (buf0, buf1)
    adb = (adb0, adb1)

    pltpu.sync_copy(zeros_hbm.at[pl.ds(s * RPT, RPT)],
                    acc.at[pl.ds(s * RPT, RPT)])
    plsc.subcore_barrier()

    iota = lax.iota(jnp.int32, 16)
    bbase = s * NBATCH

    def load_slab(bb, slot):
        pltpu.sync_copy(eidx_hbm.at[bbase + bb, 0], srcv[slot])
        pltpu.sync_copy(eidx_hbm.at[bbase + bb, 1], dstv[slot])

    def start_gathers(slot):
        pltpu.async_copy(big_hbm.at[srcv[slot]], buf[slot], semg)
        pltpu.async_copy(ad_hbm.at[dstv[slot]], adb[slot], semg)

    def wait_gathers(slot):
        pltpu.make_async_copy(big_hbm.at[srcv[slot]], buf[slot], semg).wait()
        pltpu.make_async_copy(ad_hbm.at[dstv[slot]], adb[slot], semg).wait()

    def compute(slot):
        def group(g, carry):
            g16 = pl.multiple_of(g * 16, 16)
            ev = iota + g16
            vd = dstv[slot][pl.ds(g16, 16)]
            dl = vd - off
            okm = (dl >= 0) & (dl < HALF)
            rowv[slot][pl.ds(g16, 16)] = jnp.where(okm, dl, TRASH)
            for j in range(heads):
                cj = jnp.full((16,), H + j, jnp.int32)
                va = plsc.load_gather(buf[slot], [ev, cj])
                vb = plsc.load_gather(adb[slot], [ev, jnp.full((16,), j, jnp.int32)])
                sv = va + vb
                ex = jnp.exp(jnp.where(sv > 0.0, sv, 0.2 * sv))
                plsc.store_scatter(buf[slot], [ev, cj], ex)
                for t in range(ch):
                    fv = jnp.full((16,), j * ch + t, jnp.int32)
                    vh = plsc.load_gather(buf[slot], [ev, fv])
                    plsc.store_scatter(buf[slot], [ev, fv], vh * ex)
            return carry

        lax.fori_loop(0, BATCH // 16, group, 0)

    def phase(slot, bb):
        wait_gathers(slot)
        load_slab(bb + 1, 1 - slot)
        start_gathers(1 - slot)
        compute(slot)
        pltpu.sync_copy(buf[slot], acc.at[rowv[slot]], add=True)

    load_slab(0, 0)
    start_gathers(0)

    def pair(p, carry):
        bb = p * 2
        phase(0, bb)
        phase(1, bb + 1)
        return carry

    lax.fori_loop(0, NBATCH // 2, pair, 0)
    wait_gathers(0)

    plsc.subcore_barrier()
    pltpu.sync_copy(acc.at[pl.ds(s * RPT, RPT)],
                    out_hbm.at[c, pl.ds(s * RPT, RPT)])


def _edge_sc(eidx, big_p, ad_p, zeros, heads):
    mesh = plsc.VectorSubcoreMesh(core_axis_name="c", subcore_axis_name="s",
                                  num_cores=2, num_subcores=NSUB)
    kern = pl.kernel(
        functools.partial(_edge_body, heads=heads),
        out_type=jax.ShapeDtypeStruct((2, ACC_ROWS, ROWW), jnp.float32),
        mesh=mesh,
        compiler_params=pltpu.CompilerParams(needs_layout_passes=False,
                                             use_tc_tiling_on_sc=False),
        scratch_types=(
            [pltpu.VMEM((BATCH,), jnp.int32)] * 6
            + [pltpu.VMEM((BATCH, ROWW), jnp.float32)] * 2
            + [pltpu.VMEM((BATCH, 8), jnp.float32)] * 2
            + [pltpu.VMEM_SHARED((ACC_ROWS, ROWW), jnp.float32),
               pltpu.SemaphoreType.DMA]
        ),
    )
    return kern(eidx, big_p, ad_p, zeros)


def _rep_matrix(heads):
    ch = H // heads
    rep = jnp.zeros((4, H), jnp.float32)
    for j in range(heads):
        rep = rep.at[j, j * ch:(j + 1) * ch].set(1.0)
    return rep


def _att_comb(att_s, att_d, heads, ch):
    As = jnp.zeros((H, 4), jnp.float32)
    Ad = jnp.zeros((H, 4), jnp.float32)
    for j in range(heads):
        As = As.at[j * ch:(j + 1) * ch, j].set(att_s[j])
        Ad = Ad.at[j * ch:(j + 1) * ch, j].set(att_d[j])
    return As, Ad


def kernel(agent_hist, lane_nodes, edge_index_aa, edge_index_al, params):
    p = params
    f32 = jnp.float32

    def prep_lstm(x, Din):
        T = x.shape[1]
        xt = jnp.transpose(x, (1, 0, 2))
        return jnp.pad(xt, ((0, 0), (0, 0), (0, 8 - Din)))

    ah = prep_lstm(agent_hist, 5)
    ln = prep_lstm(lane_nodes, 2)
    aw_ih = jnp.pad(p['agent_Wih'].T, ((0, 3), (0, 0)))
    lw_ih = jnp.pad(p['lane_Wih'].T, ((0, 6), (0, 0)))
    ab = (p['agent_bih'] + p['agent_bhh'])[None, :]
    lb = (p['lane_bih'] + p['lane_bhh'])[None, :]
    agent_emb = _lstm(ah, aw_ih, p['agent_Whh'].T, ab, mean=False)
    lane_emb = _lstm(ln, lw_ih, p['lane_Whh'].T, lb, mean=True)

    E_ALLOC = (NBT + 1) * BATCH
    epad = jnp.full((E_ALLOC - E_EDGES,), N_NODES, jnp.int32)

    def pack_edges(src, dst):
        srcp = jnp.concatenate([src.astype(jnp.int32), epad]).reshape(-1, BATCH)
        dstp = jnp.concatenate([dst.astype(jnp.int32), epad]).reshape(-1, BATCH)
        return jnp.stack([srcp, dstp], axis=1)

    eidx_aa = pack_edges(edge_index_aa[0], edge_index_aa[1])
    eidx_al = pack_edges(edge_index_al[1], edge_index_al[0])
    zeros_acc = jnp.zeros((ACC_ROWS, ROWW), f32)

    def pad_tab(a):
        return jnp.pad(a, ((0, N_PAD - N_NODES), (0, 0)))

    def pad_ad(a):
        return jnp.pad(a, ((0, N_PAD - N_NODES), (0, 4)))

    one = jnp.ones((1, H), f32)
    zero = jnp.zeros((1, H), f32)
    bn_g_aa = (p['aa_bn_gamma'] / jnp.sqrt(1.0 + 1e-5))[None, :]
    bn_b_aa = p['aa_bn_beta'][None, :]
    bn_g_al = (p['al_bn_gamma'] / jnp.sqrt(1.0 + 1e-5))[None, :]
    bn_b_al = p['al_bn_beta'][None, :]

    NB0 = HALF // PBLK
    NB1 = (N_NODES - HALF) // PBLK

    def post_both(accs, big, adt, x_dst, Wres, b, heads, gamma, beta, act):
        rep = _rep_matrix(heads)
        o0 = _gat_post(accs[0], big, adt, x_dst, Wres, b[None, :], rep,
                       gamma, beta, heads, act, NB0, 0)
        o1 = _gat_post(accs[1], big, adt, x_dst, Wres, b[None, :], rep,
                       gamma, beta, heads, act, NB1, NB0)
        return jnp.concatenate([o0, o1], axis=0)

    def gat_same(x, W, att_s, att_d, Wres, b, heads, eidx, gamma, beta, act):
        ch = H // heads
        As, Ad = _att_comb(att_s, att_d, heads, ch)
        big, adt = _gat_pre(x, W, As, Ad)
        accs = _edge_sc(eidx, pad_tab(big), pad_ad(adt), zeros_acc, heads)
        return post_both(accs, big, adt, x, Wres, b, heads, gamma, beta, act)

    x = gat_same(agent_emb, p['aa0_W'], p['aa0_att_src'], p['aa0_att_dst'],
                 p['aa0_Wres'], p['aa0_b'], 4, eidx_aa,
                 bn_g_aa, bn_b_aa, "elu_bn")
    agent_social = gat_same(x, p['aa1_W'], p['aa1_att_src'], p['aa1_att_dst'],
                            p['aa1_Wres'], p['aa1_b'], 4, eidx_aa,
                            one, zero, "none")
    As, Ad = _att_comb(p['al0_att_src'], p['al0_att_dst'], 2, 32)
    big, adt = _gat_pre2(lane_emb, agent_social, p['al0_Wsrc'], p['al0_Wdst'],
                         As, Ad)
    accs = _edge_sc(eidx_al, pad_tab(big), pad_ad(adt), zeros_acc, 2)
    y = post_both(accs, big, adt, agent_social, p['al0_Wres'], p['al0_b'],
                  2, bn_g_al, bn_b_al, "relu_bn")
    agent_map = gat_same(y, p['al1_W'], p['al1_att_src'], p['al1_att_dst'],
                         p['al1_Wres'], p['al1_b'], 2, eidx_al,
                         one, zero, "none")

    return (agent_emb, agent_social, agent_map, lane_emb)


# speedup vs baseline: 43.2135x; 1.0783x over previous
import functools

import jax
import jax.numpy as jnp
from jax import lax
from jax.experimental import pallas as pl
from jax.experimental.pallas import tpu as pltpu
from jax.experimental.pallas import tpu_sc as plsc

H = 64
N_NODES = 50000
BLK = 2000

E_EDGES = 800000
BATCH = 96
NSUB = 16
NBATCH = 522
EPT = BATCH * NBATCH
E_PAD = EPT * NSUB
NBT = E_PAD // BATCH
N_PAD = 50008
HALF = 25000
ACC_ROWS = 25088
RPT = ACC_ROWS // NSUB
TRASH = ACC_ROWS - 1
ROWW = 72
PBLK = 1000


def _lstm_body(x_ref, wih_ref, whh_ref, b_ref, out_ref, *, T, mean):
    B = x_ref.shape[1]
    h0 = jnp.zeros((B, H), jnp.float32)

    def step(t, carry):
        h, c, acc = carry
        xt = x_ref[t]
        gates = jnp.dot(xt, wih_ref[...], preferred_element_type=jnp.float32)
        gates += jnp.dot(h, whh_ref[...], preferred_element_type=jnp.float32)
        gates += b_ref[...]
        i = jax.nn.sigmoid(gates[:, 0:H])
        f = jax.nn.sigmoid(gates[:, H:2 * H])
        g = jnp.tanh(gates[:, 2 * H:3 * H])
        o = jax.nn.sigmoid(gates[:, 3 * H:4 * H])
        c = f * c + i * g
        h = o * jnp.tanh(c)
        return h, c, acc + h

    h, c, acc = lax.fori_loop(0, T, step, (h0, h0, h0))
    out_ref[...] = (acc * (1.0 / T)) if mean else h


def _lstm(x, Wih, Whh, bias, mean):
    T, N, Dp = x.shape
    grid = (N // BLK,)
    return pl.pallas_call(
        functools.partial(_lstm_body, T=T, mean=mean),
        grid=grid,
        in_specs=[
            pl.BlockSpec((T, BLK, Dp), lambda i: (0, i, 0)),
            pl.BlockSpec((Dp, 4 * H), lambda i: (0, 0)),
            pl.BlockSpec((H, 4 * H), lambda i: (0, 0)),
            pl.BlockSpec((1, 4 * H), lambda i: (0, 0)),
        ],
        out_specs=pl.BlockSpec((BLK, H), lambda i: (i, 0)),
        out_shape=jax.ShapeDtypeStruct((N, H), jnp.float32),
    )(x, Wih, Whh, bias)


def _pre_body(x_ref, w_ref, as_ref, ad_ref, big_ref, adt_ref):
    hs = jnp.dot(x_ref[...], w_ref[...], preferred_element_type=jnp.float32)
    a_s = jnp.dot(hs, as_ref[...], preferred_element_type=jnp.float32)
    B = hs.shape[0]
    big_ref[...] = jnp.concatenate(
        [hs, a_s, jnp.zeros((B, ROWW - H - 4), jnp.float32)], axis=1)
    adt_ref[...] = jnp.dot(hs, ad_ref[...], preferred_element_type=jnp.float32)


def _gat_pre(x, W, As, Ad):
    N = x.shape[0]
    return pl.pallas_call(
        _pre_body,
        grid=(N // BLK,),
        in_specs=[
            pl.BlockSpec((BLK, H), lambda i: (i, 0)),
            pl.BlockSpec((H, H), lambda i: (0, 0)),
            pl.BlockSpec((H, 4), lambda i: (0, 0)),
            pl.BlockSpec((H, 4), lambda i: (0, 0)),
        ],
        out_specs=[
            pl.BlockSpec((BLK, ROWW), lambda i: (i, 0)),
            pl.BlockSpec((BLK, 4), lambda i: (i, 0)),
        ],
        out_shape=[
            jax.ShapeDtypeStruct((N, ROWW), jnp.float32),
            jax.ShapeDtypeStruct((N, 4), jnp.float32),
        ],
    )(x, W, As, Ad)


def _pre2_body(xs_ref, xd_ref, ws_ref, wd_ref, as_ref, ad_ref,
               big_ref, adt_ref):
    hs = jnp.dot(xs_ref[...], ws_ref[...], preferred_element_type=jnp.float32)
    hd = jnp.dot(xd_ref[...], wd_ref[...], preferred_element_type=jnp.float32)
    a_s = jnp.dot(hs, as_ref[...], preferred_element_type=jnp.float32)
    B = hs.shape[0]
    big_ref[...] = jnp.concatenate(
        [hs, a_s, jnp.zeros((B, ROWW - H - 4), jnp.float32)], axis=1)
    adt_ref[...] = jnp.dot(hd, ad_ref[...], preferred_element_type=jnp.float32)


def _gat_pre2(x_src, x_dst, Wsrc, Wdst, As, Ad):
    N = x_src.shape[0]
    return pl.pallas_call(
        _pre2_body,
        grid=(N // BLK,),
        in_specs=[
            pl.BlockSpec((BLK, H), lambda i: (i, 0)),
            pl.BlockSpec((BLK, H), lambda i: (i, 0)),
            pl.BlockSpec((H, H), lambda i: (0, 0)),
            pl.BlockSpec((H, H), lambda i: (0, 0)),
            pl.BlockSpec((H, 4), lambda i: (0, 0)),
            pl.BlockSpec((H, 4), lambda i: (0, 0)),
        ],
        out_specs=[
            pl.BlockSpec((BLK, ROWW), lambda i: (i, 0)),
            pl.BlockSpec((BLK, 4), lambda i: (i, 0)),
        ],
        out_shape=[
            jax.ShapeDtypeStruct((N, ROWW), jnp.float32),
            jax.ShapeDtypeStruct((N, 4), jnp.float32),
        ],
    )(x_src, x_dst, Wsrc, Wdst, As, Ad)


def _post_body(acc_ref, big_ref, adt_ref, xd_ref, wres_ref, b_ref, rep_ref,
               g_ref, be_ref, *, heads, act):
    hs_vals = big_ref[:, 0:H]
    a_s = big_ref[:, H:H + 4]
    a_d = adt_ref[...]
    s = a_s + a_d
    exii = jnp.exp(jnp.where(s > 0, s, 0.2 * s))
    exf = jnp.dot(exii[:, 0:4], rep_ref[...],
                  preferred_element_type=jnp.float32)
    num = acc_ref[:, 0:H]
    den = jnp.dot(acc_ref[:, H:H + 4], rep_ref[...],
                  preferred_element_type=jnp.float32)
    num = num + exf * hs_vals
    den = den + exf
    o = num / (den + 1e-16)
    o = o + jnp.dot(xd_ref[...], wres_ref[...],
                    preferred_element_type=jnp.float32) + b_ref[...]
    if act == "elu_bn":
        o = g_ref[...] * o + be_ref[...]
        o = jnp.where(o > 0, o, jnp.exp(jnp.minimum(o, 0.0)) - 1.0)
    elif act == "relu_bn":
        o = g_ref[...] * o + be_ref[...]
        o = jnp.maximum(o, 0.0)
    return o


def _post_wrap(acc_ref, big_ref, adt_ref, xd_ref, wres_ref, b_ref, rep_ref,
               g_ref, be_ref, out_ref, *, heads, act):
    out_ref[...] = _post_body(acc_ref, big_ref, adt_ref, xd_ref, wres_ref,
                              b_ref, rep_ref, g_ref, be_ref,
                              heads=heads, act=act)


def _gat_post(acc, big, adt, x_dst, Wres, b, rep, gamma, beta, heads, act,
              nblocks, row_off):
    return pl.pallas_call(
        functools.partial(_post_wrap, heads=heads, act=act),
        grid=(nblocks,),
        in_specs=[
            pl.BlockSpec((PBLK, ROWW), lambda i: (i, 0)),
            pl.BlockSpec((PBLK, ROWW), lambda i: (i + row_off, 0)),
            pl.BlockSpec((PBLK, 4), lambda i: (i + row_off, 0)),
            pl.BlockSpec((PBLK, H), lambda i: (i + row_off, 0)),
            pl.BlockSpec((H, H), lambda i: (0, 0)),
            pl.BlockSpec((1, H), lambda i: (0, 0)),
            pl.BlockSpec((4, H), lambda i: (0, 0)),
            pl.BlockSpec((1, H), lambda i: (0, 0)),
            pl.BlockSpec((1, H), lambda i: (0, 0)),
        ],
        out_specs=pl.BlockSpec((PBLK, H), lambda i: (i, 0)),
        out_shape=jax.ShapeDtypeStruct((nblocks * PBLK, H), jnp.float32),
    )(acc, big, adt, x_dst, Wres, b, rep, gamma, beta)


def _edge_body(eidx_hbm, big_hbm, ad_hbm, zeros_hbm, out_hbm,
               slab0, slab1, srcv0, srcv1, dstv0, dstv1, rowv0, rowv1,
               buf0, buf1, adb0, adb1,
               acc, semg, *, heads):
    ch = H // heads
    c = lax.axis_index("c")
    s = lax.axis_index("s")
    off = c * HALF
    slab = (slab0, slab1)
    srcv = (srcv0, srcv1)
    dstv = (dstv0, dstv1)
    rowv = (rowv0, rowv1)
    buf = (buf0, buf1)
    adb = (adb0, adb1)

    pltpu.sync_copy(zeros_hbm.at[pl.ds(s * RPT, RPT)],
                    acc.at[pl.ds(s * RPT, RPT)])
    plsc.subcore_barrier()

    iota = lax.iota(jnp.int32, 16)
    bbase = s * NBATCH

    def load_slab(bb, slot):
        pltpu.sync_copy(eidx_hbm.at[bbase + bb], slab[slot])
        for g in range(BATCH // 16):
            sl = pl.ds(g * 16, 16)
            srcv[slot][sl] = slab[slot][0, sl]
            dstv[slot][sl] = slab[slot][1, sl]

    def start_gathers(slot):
        pltpu.async_copy(big_hbm.at[srcv[slot]], buf[slot], semg)
        pltpu.async_copy(ad_hbm.at[dstv[slot]], adb[slot], semg)

    def wait_gathers(slot):
        pltpu.make_async_copy(big_hbm.at[srcv[slot]], buf[slot], semg).wait()
        pltpu.make_async_copy(ad_hbm.at[dstv[slot]], adb[slot], semg).wait()


    def compute(slot):
        def group(g, carry):
            g16 = pl.multiple_of(g * 16, 16)
            ev = iota + g16
            vd = dstv[slot][pl.ds(g16, 16)]
            dl = vd - off
            okm = (dl >= 0) & (dl < HALF)
            rowv[slot][pl.ds(g16, 16)] = jnp.where(okm, dl, TRASH)
            for j in range(heads):
                cj = jnp.full((16,), H + j, jnp.int32)
                va = plsc.load_gather(buf[slot], [ev, cj])
                vb = plsc.load_gather(adb[slot], [ev, jnp.full((16,), j, jnp.int32)])
                sv = va + vb
                ex = jnp.exp(jnp.where(sv > 0.0, sv, 0.2 * sv))
                plsc.store_scatter(buf[slot], [ev, cj], ex)
                for t in range(ch):
                    fv = jnp.full((16,), j * ch + t, jnp.int32)
                    vh = plsc.load_gather(buf[slot], [ev, fv])
                    plsc.store_scatter(buf[slot], [ev, fv], vh * ex)
            return carry

        lax.fori_loop(0, BATCH // 16, group, 0)

    def phase(slot, bb):
        wait_gathers(slot)
        load_slab(bb + 1, 1 - slot)
        start_gathers(1 - slot)
        compute(slot)
        pltpu.sync_copy(buf[slot], acc.at[rowv[slot]], add=True)

    load_slab(0, 0)
    start_gathers(0)

    def pair(p, carry):
        bb = p * 2
        phase(0, bb)
        phase(1, bb + 1)
        return carry

    lax.fori_loop(0, NBATCH // 2, pair, 0)
    wait_gathers(0)

    plsc.subcore_barrier()
    pltpu.sync_copy(acc.at[pl.ds(s * RPT, RPT)],
                    out_hbm.at[c, pl.ds(s * RPT, RPT)])


def _edge_sc(eidx, big_p, ad_p, zeros, heads):
    mesh = plsc.VectorSubcoreMesh(core_axis_name="c", subcore_axis_name="s",
                                  num_cores=2, num_subcores=NSUB)
    kern = pl.kernel(
        functools.partial(_edge_body, heads=heads),
        out_type=jax.ShapeDtypeStruct((2, ACC_ROWS, ROWW), jnp.float32),
        mesh=mesh,
        compiler_params=pltpu.CompilerParams(needs_layout_passes=False,
                                             use_tc_tiling_on_sc=False),
        scratch_types=(
            [pltpu.VMEM((2, BATCH), jnp.int32)] * 2
            + [pltpu.VMEM((BATCH,), jnp.int32)] * 6
            + [pltpu.VMEM((BATCH, ROWW), jnp.float32)] * 2
            + [pltpu.VMEM((BATCH, 8), jnp.float32)] * 2
            + [pltpu.VMEM_SHARED((ACC_ROWS, ROWW), jnp.float32),
               pltpu.SemaphoreType.DMA]
        ),
    )
    return kern(eidx, big_p, ad_p, zeros)


def _rep_matrix(heads):
    ch = H // heads
    rep = jnp.zeros((4, H), jnp.float32)
    for j in range(heads):
        rep = rep.at[j, j * ch:(j + 1) * ch].set(1.0)
    return rep


def _att_comb(att_s, att_d, heads, ch):
    As = jnp.zeros((H, 4), jnp.float32)
    Ad = jnp.zeros((H, 4), jnp.float32)
    for j in range(heads):
        As = As.at[j * ch:(j + 1) * ch, j].set(att_s[j])
        Ad = Ad.at[j * ch:(j + 1) * ch, j].set(att_d[j])
    return As, Ad


def kernel(agent_hist, lane_nodes, edge_index_aa, edge_index_al, params):
    p = params
    f32 = jnp.float32

    def prep_lstm(x, Din):
        T = x.shape[1]
        xt = jnp.transpose(x, (1, 0, 2))
        return jnp.pad(xt, ((0, 0), (0, 0), (0, 8 - Din)))

    ah = prep_lstm(agent_hist, 5)
    ln = prep_lstm(lane_nodes, 2)
    aw_ih = jnp.pad(p['agent_Wih'].T, ((0, 3), (0, 0)))
    lw_ih = jnp.pad(p['lane_Wih'].T, ((0, 6), (0, 0)))
    ab = (p['agent_bih'] + p['agent_bhh'])[None, :]
    lb = (p['lane_bih'] + p['lane_bhh'])[None, :]
    agent_emb = _lstm(ah, aw_ih, p['agent_Whh'].T, ab, mean=False)
    lane_emb = _lstm(ln, lw_ih, p['lane_Whh'].T, lb, mean=True)

    E_ALLOC = (NBT + 1) * BATCH
    epad = jnp.full((E_ALLOC - E_EDGES,), N_NODES, jnp.int32)

    def pack_edges(src, dst):
        srcp = jnp.concatenate([src.astype(jnp.int32), epad]).reshape(-1, BATCH)
        dstp = jnp.concatenate([dst.astype(jnp.int32), epad]).reshape(-1, BATCH)
        return jnp.stack([srcp, dstp], axis=1)

    eidx_aa = pack_edges(edge_index_aa[0], edge_index_aa[1])
    eidx_al = pack_edges(edge_index_al[1], edge_index_al[0])
    zeros_acc = jnp.zeros((ACC_ROWS, ROWW), f32)

    def pad_tab(a):
        return jnp.pad(a, ((0, N_PAD - N_NODES), (0, 0)))

    def pad_ad(a):
        return jnp.pad(a, ((0, N_PAD - N_NODES), (0, 4)))

    one = jnp.ones((1, H), f32)
    zero = jnp.zeros((1, H), f32)
    bn_g_aa = (p['aa_bn_gamma'] / jnp.sqrt(1.0 + 1e-5))[None, :]
    bn_b_aa = p['aa_bn_beta'][None, :]
    bn_g_al = (p['al_bn_gamma'] / jnp.sqrt(1.0 + 1e-5))[None, :]
    bn_b_al = p['al_bn_beta'][None, :]

    NB0 = HALF // PBLK
    NB1 = (N_NODES - HALF) // PBLK

    def post_both(accs, big, adt, x_dst, Wres, b, heads, gamma, beta, act):
        rep = _rep_matrix(heads)
        o0 = _gat_post(accs[0], big, adt, x_dst, Wres, b[None, :], rep,
                       gamma, beta, heads, act, NB0, 0)
        o1 = _gat_post(accs[1], big, adt, x_dst, Wres, b[None, :], rep,
                       gamma, beta, heads, act, NB1, NB0)
        return jnp.concatenate([o0, o1], axis=0)

    def gat_same(x, W, att_s, att_d, Wres, b, heads, eidx, gamma, beta, act):
        ch = H // heads
        As, Ad = _att_comb(att_s, att_d, heads, ch)
        big, adt = _gat_pre(x, W, As, Ad)
        accs = _edge_sc(eidx, pad_tab(big), pad_ad(adt), zeros_acc, heads)
        return post_both(accs, big, adt, x, Wres, b, heads, gamma, beta, act)

    x = gat_same(agent_emb, p['aa0_W'], p['aa0_att_src'], p['aa0_att_dst'],
                 p['aa0_Wres'], p['aa0_b'], 4, eidx_aa,
                 bn_g_aa, bn_b_aa, "elu_bn")
    agent_social = gat_same(x, p['aa1_W'], p['aa1_att_src'], p['aa1_att_dst'],
                            p['aa1_Wres'], p['aa1_b'], 4, eidx_aa,
                            one, zero, "none")
    As, Ad = _att_comb(p['al0_att_src'], p['al0_att_dst'], 2, 32)
    big, adt = _gat_pre2(lane_emb, agent_social, p['al0_Wsrc'], p['al0_Wdst'],
                         As, Ad)
    accs = _edge_sc(eidx_al, pad_tab(big), pad_ad(adt), zeros_acc, 2)
    y = post_both(accs, big, adt, agent_social, p['al0_Wres'], p['al0_b'],
                  2, bn_g_al, bn_b_al, "relu_bn")
    agent_map = gat_same(y, p['al1_W'], p['al1_att_src'], p['al1_att_dst'],
                         p['al1_Wres'], p['al1_b'], 2, eidx_al,
                         one, zero, "none")

    return (agent_emb, agent_social, agent_map, lane_emb)
